# Initial kernel scaffold; baseline (speedup 1.0000x reference)
import jax
import jax.numpy as jnp
from jax.experimental import pallas as pl


def kernel(x, distance, edge_index, W0, b0, W1, b1, Wih, Whh, bih, bhh, Wf, bf):
    raise NotImplementedError("write your pallas kernel here")


# speedup vs baseline: 13.0436x; 13.0436x over previous
import functools

import jax
import jax.numpy as jnp
from jax import lax
from jax.experimental import pallas as pl
from jax.experimental.pallas import tpu as pltpu
from jax.experimental.pallas import tpu_sc as plsc

B, T, L, H = 16, 128, 128, 128
N = B * T
E = 65536
OUT = 12 * L
NC, NS = 2, 16
NW = NC * NS
EPT = E // NW
CH = 128
NCH = EPT // CH
RPT = N // NS


def _deg_body(dst_hbm, dp_hbm, dstv, degv):
    wid = lax.axis_index("s") * NC + lax.axis_index("c")
    pltpu.sync_copy(dst_hbm.at[wid], dstv)

    def zero(i, carry):
        degv[pl.ds(i * 16, 16)] = jnp.zeros((16,), jnp.float32)
        return carry
    lax.fori_loop(0, N // 16, zero, 0)

    ones = jnp.ones((16,), jnp.float32)

    def count(i, carry):
        idx = dstv[pl.ds(i * 16, 16)]
        plsc.addupdate_scatter(degv, [idx], ones)
        return carry
    lax.fori_loop(0, EPT // 16, count, 0)

    pltpu.sync_copy(degv, dp_hbm.at[wid])


def _scatter_body(g_hbm, src_hbm, dst_hbm, zero_hbm, part_hbm,
                  srcv, dstv, rows, acc, gsem):
    cid = lax.axis_index("c")
    sid = lax.axis_index("s")
    wid = sid * NC + cid
    pltpu.sync_copy(zero_hbm, acc.at[pl.ds(sid * RPT, RPT)])
    pltpu.sync_copy(src_hbm.at[wid], srcv)
    pltpu.sync_copy(dst_hbm.at[wid], dstv)
    plsc.subcore_barrier()
    for j in range(NCH):
        pltpu.async_copy(g_hbm.at[srcv.at[j]], rows, gsem).wait()
        pltpu.sync_copy(rows, acc.at[dstv.at[j]], add=True)
    plsc.subcore_barrier()
    pltpu.sync_copy(acc.at[pl.ds(sid * RPT, RPT)],
                    part_hbm.at[cid, pl.ds(sid * RPT, RPT)])


def _proj_body(x_ref, d_ref, w0b_ref, w0a_ref, dpt_ref, g0_ref, dinv_ref):
    c0 = jnp.dot(d_ref[...], w0b_ref[...], preferred_element_type=jnp.float32)
    hw0 = jnp.dot(x_ref[...], w0a_ref[...],
                  preferred_element_type=jnp.float32) + c0
    deg = jnp.sum(dpt_ref[...], axis=1, keepdims=True) + 1.0
    dinv = lax.rsqrt(deg)
    dinv_ref[...] = dinv
    g0_ref[...] = hw0 * dinv


def _layer_body(s0_ref, s1_ref, g_ref, dinv_ref, b_ref, w_ref, out_ref):
    dinv = dinv_ref[...]
    h = jnp.maximum(
        dinv * (s0_ref[...] + s1_ref[...] + g_ref[...]) + b_ref[...], 0.0)
    out_ref[...] = jnp.dot(
        h, w_ref[...], preferred_element_type=jnp.float32) * dinv


def _lstm_in_body(s0_ref, s1_ref, g_ref, dinv_ref, b_ref, wih_ref,
                  bih_ref, bhh_ref, xg_ref):
    h2 = jnp.maximum(
        dinv_ref[...] * (s0_ref[...] + s1_ref[...] + g_ref[...]) + b_ref[...],
        0.0)
    xg_ref[...] = (jnp.dot(h2, wih_ref[...], preferred_element_type=jnp.float32)
                   + bih_ref[...] + bhh_ref[...])


def _lstm_body(xg_ref, whh_ref, wf_ref, bf_ref, out_ref):
    whh = whh_ref[...]

    def step(t, carry):
        hp, cp = carry
        gates = xg_ref[t] + jnp.dot(hp, whh,
                                    preferred_element_type=jnp.float32)
        i = jax.nn.sigmoid(gates[:, :H])
        f = jax.nn.sigmoid(gates[:, H:2 * H])
        g = jnp.tanh(gates[:, 2 * H:3 * H])
        o = jax.nn.sigmoid(gates[:, 3 * H:])
        c = f * cp + i * g
        hn = o * jnp.tanh(c)
        return (hn, c)

    init = (jnp.zeros((B, H), jnp.float32), jnp.zeros((B, H), jnp.float32))
    hn, _ = lax.fori_loop(0, T, step, init)
    out_ref[...] = jnp.dot(
        hn, wf_ref[...], preferred_element_type=jnp.float32) + bf_ref[...]


def _tc(body, out_shapes):
    return pl.pallas_call(body, out_shape=out_shapes)


@functools.cache
def _sc_kernels():
    mesh = plsc.VectorSubcoreMesh(
        core_axis_name="c", subcore_axis_name="s",
        num_cores=NC, num_subcores=NS)
    params = pltpu.CompilerParams(needs_layout_passes=False)
    deg = pl.kernel(
        _deg_body,
        out_type=jax.ShapeDtypeStruct((NW, N), jnp.float32),
        mesh=mesh,
        compiler_params=params,
        scratch_types=[
            pltpu.VMEM((EPT,), jnp.int32),
            pltpu.VMEM((N,), jnp.float32),
        ],
    )
    scatter = pl.kernel(
        _scatter_body,
        out_type=jax.ShapeDtypeStruct((NC, N, H), jnp.float32),
        mesh=mesh,
        compiler_params=params,
        scratch_types=[
            pltpu.VMEM((NCH, CH), jnp.int32),
            pltpu.VMEM((NCH, CH), jnp.int32),
            pltpu.VMEM((CH, H), jnp.float32),
            pltpu.VMEM_SHARED((N, H), jnp.float32),
            pltpu.SemaphoreType.DMA,
        ],
    )
    return deg, scatter


def kernel(x, distance, edge_index, W0, b0, W1, b1, Wih, Whh, bih, bhh, Wf, bf):
    x2d = x.reshape(N, L)
    dflat = distance.reshape(1, L * L)
    W0a = W0[:L]
    W0b = W0[L:]
    src_r = edge_index[0].reshape(NW, NCH, CH)
    dst_r = edge_index[1].reshape(NW, NCH, CH)
    dst_flat = edge_index[1].reshape(NW, EPT)
    zero_rows = jnp.zeros((RPT, H), jnp.float32)
    _deg_kernel, _scatter_kernel = _sc_kernels()

    dp = _deg_kernel(dst_flat)
    dpt = dp.T

    g0, dinv = _tc(_proj_body, [
        jax.ShapeDtypeStruct((N, H), jnp.float32),
        jax.ShapeDtypeStruct((N, 1), jnp.float32),
    ])(x2d, dflat, W0b, W0a, dpt)

    s0 = _scatter_kernel(g0, src_r, dst_r, zero_rows)
    g1 = _tc(_layer_body, jax.ShapeDtypeStruct((N, H), jnp.float32))(
        s0[0], s0[1], g0, dinv, b0.reshape(1, H), W1)

    s1 = _scatter_kernel(g1, src_r, dst_r, zero_rows)
    xg = _tc(_lstm_in_body, jax.ShapeDtypeStruct((N, 4 * H), jnp.float32))(
        s1[0], s1[1], g1, dinv, b1.reshape(1, H), Wih.T,
        bih.reshape(1, 4 * H), bhh.reshape(1, 4 * H))

    xg3 = xg.reshape(B, T, 4 * H).transpose(1, 0, 2)
    out = _tc(_lstm_body, jax.ShapeDtypeStruct((B, OUT), jnp.float32))(
        xg3, Whh.T, Wf, bf.reshape(1, OUT))
    return out.reshape(B, 12, L)


# speedup vs baseline: 15.2713x; 1.1708x over previous
import functools

import jax
import jax.numpy as jnp
from jax import lax
from jax.experimental import pallas as pl
from jax.experimental.pallas import tpu as pltpu
from jax.experimental.pallas import tpu_sc as plsc

B, T, L, H = 16, 128, 128, 128
N = B * T
E = 65536
OUT = 12 * L
NC, NS = 2, 16
NW = NC * NS
EPT = E // NW
CH = 128
NCH = EPT // CH
RPT = N // NS


def _deg_body(dst_hbm, dp_hbm, dstv, degv):
    wid = lax.axis_index("s") * NC + lax.axis_index("c")
    pltpu.sync_copy(dst_hbm.at[wid], dstv)

    def zero(i, carry):
        degv[pl.ds(i * 16, 16)] = jnp.zeros((16,), jnp.float32)
        return carry
    lax.fori_loop(0, N // 16, zero, 0)

    ones = jnp.ones((16,), jnp.float32)

    def count(i, carry):
        idx = dstv[pl.ds(i * 16, 16)]
        plsc.addupdate_scatter(degv, [idx], ones)
        return carry
    lax.fori_loop(0, EPT // 16, count, 0)

    pltpu.sync_copy(degv, dp_hbm.at[wid])


def _scatter_body(g_hbm, src_hbm, dst_hbm, zero_hbm, part_hbm,
                  srcv, dstv, rows0, rows1, acc, gsem0, gsem1):
    cid = lax.axis_index("c")
    sid = lax.axis_index("s")
    wid = sid * NC + cid
    pltpu.sync_copy(zero_hbm, acc.at[pl.ds(sid * RPT, RPT)])
    pltpu.sync_copy(src_hbm.at[wid], srcv)
    pltpu.sync_copy(dst_hbm.at[wid], dstv)
    plsc.subcore_barrier()
    bufs = (rows0, rows1)
    sems = (gsem0, gsem1)
    descs = [None, None]
    descs[0] = pltpu.async_copy(g_hbm.at[srcv.at[0]], rows0, gsem0)
    for j in range(NCH):
        b = j % 2
        descs[b].wait()
        if j + 1 < NCH:
            descs[1 - b] = pltpu.async_copy(
                g_hbm.at[srcv.at[j + 1]], bufs[1 - b], sems[1 - b])
        pltpu.sync_copy(bufs[b], acc.at[dstv.at[j]], add=True)
    plsc.subcore_barrier()
    pltpu.sync_copy(acc.at[pl.ds(sid * RPT, RPT)],
                    part_hbm.at[cid, pl.ds(sid * RPT, RPT)])


def _proj_body(x_ref, d_ref, w0b_ref, w0a_ref, dpt_ref, g0_ref, dinv_ref):
    c0 = jnp.dot(d_ref[...], w0b_ref[...], preferred_element_type=jnp.float32)
    hw0 = jnp.dot(x_ref[...], w0a_ref[...],
                  preferred_element_type=jnp.float32) + c0
    deg = jnp.sum(dpt_ref[...], axis=1, keepdims=True) + 1.0
    dinv = lax.rsqrt(deg)
    dinv_ref[...] = dinv
    g0_ref[...] = hw0 * dinv


def _layer_body(s_ref, g_ref, dinv_ref, b_ref, w_ref, out_ref):
    dinv = dinv_ref[...]
    h = jnp.maximum(
        dinv * (s_ref[0] + s_ref[1] + g_ref[...]) + b_ref[...], 0.0)
    out_ref[...] = jnp.dot(
        h, w_ref[...], preferred_element_type=jnp.float32) * dinv


def _lstm_in_body(s_ref, g_ref, dinv_ref, b_ref, wih_ref,
                  bih_ref, bhh_ref, xg_ref):
    h2 = jnp.maximum(
        dinv_ref[...] * (s_ref[0] + s_ref[1] + g_ref[...]) + b_ref[...],
        0.0)
    xg_ref[...] = (jnp.dot(h2, wih_ref[...], preferred_element_type=jnp.float32)
                   + bih_ref[...] + bhh_ref[...])


def _lstm_body(xg_ref, whh_ref, wf_ref, bf_ref, out_ref):
    whh = whh_ref[...]

    def step(t, carry):
        hp, cp = carry
        gates = xg_ref[t] + jnp.dot(hp, whh,
                                    preferred_element_type=jnp.float32)
        i = jax.nn.sigmoid(gates[:, :H])
        f = jax.nn.sigmoid(gates[:, H:2 * H])
        g = jnp.tanh(gates[:, 2 * H:3 * H])
        o = jax.nn.sigmoid(gates[:, 3 * H:])
        c = f * cp + i * g
        hn = o * jnp.tanh(c)
        return (hn, c)

    init = (jnp.zeros((B, H), jnp.float32), jnp.zeros((B, H), jnp.float32))
    hn, _ = lax.fori_loop(0, T, step, init)
    out_ref[...] = jnp.dot(
        hn, wf_ref[...], preferred_element_type=jnp.float32) + bf_ref[...]


def _tc(body, out_shapes):
    return pl.pallas_call(body, out_shape=out_shapes)


@functools.cache
def _sc_kernels():
    mesh = plsc.VectorSubcoreMesh(
        core_axis_name="c", subcore_axis_name="s",
        num_cores=NC, num_subcores=NS)
    params = pltpu.CompilerParams(needs_layout_passes=False)
    deg = pl.kernel(
        _deg_body,
        out_type=jax.ShapeDtypeStruct((NW, N), jnp.float32),
        mesh=mesh,
        compiler_params=params,
        scratch_types=[
            pltpu.VMEM((EPT,), jnp.int32),
            pltpu.VMEM((N,), jnp.float32),
        ],
    )
    scatter = pl.kernel(
        _scatter_body,
        out_type=jax.ShapeDtypeStruct((NC, N, H), jnp.float32),
        mesh=mesh,
        compiler_params=params,
        scratch_types=[
            pltpu.VMEM((NCH, CH), jnp.int32),
            pltpu.VMEM((NCH, CH), jnp.int32),
            pltpu.VMEM((CH, H), jnp.float32),
            pltpu.VMEM((CH, H), jnp.float32),
            pltpu.VMEM_SHARED((N, H), jnp.float32),
            pltpu.SemaphoreType.DMA,
            pltpu.SemaphoreType.DMA,
        ],
    )
    return deg, scatter


def kernel(x, distance, edge_index, W0, b0, W1, b1, Wih, Whh, bih, bhh, Wf, bf):
    x2d = x.reshape(N, L)
    dflat = distance.reshape(1, L * L)
    W0a = W0[:L]
    W0b = W0[L:]
    src_r = edge_index[0].reshape(NW, NCH, CH)
    dst_r = edge_index[1].reshape(NW, NCH, CH)
    dst_flat = edge_index[1].reshape(NW, EPT)
    zero_rows = jnp.zeros((RPT, H), jnp.float32)
    _deg_kernel, _scatter_kernel = _sc_kernels()

    dp = _deg_kernel(dst_flat)
    dpt = dp.T

    g0, dinv = _tc(_proj_body, [
        jax.ShapeDtypeStruct((N, H), jnp.float32),
        jax.ShapeDtypeStruct((N, 1), jnp.float32),
    ])(x2d, dflat, W0b, W0a, dpt)

    s0 = _scatter_kernel(g0, src_r, dst_r, zero_rows)
    g1 = _tc(_layer_body, jax.ShapeDtypeStruct((N, H), jnp.float32))(
        s0, g0, dinv, b0.reshape(1, H), W1)

    s1 = _scatter_kernel(g1, src_r, dst_r, zero_rows)
    xg = _tc(_lstm_in_body, jax.ShapeDtypeStruct((N, 4 * H), jnp.float32))(
        s1, g1, dinv, b1.reshape(1, H), Wih.T,
        bih.reshape(1, 4 * H), bhh.reshape(1, 4 * H))

    xg3 = xg.reshape(B, T, 4 * H).transpose(1, 0, 2)
    out = _tc(_lstm_body, jax.ShapeDtypeStruct((B, OUT), jnp.float32))(
        xg3, Whh.T, Wf, bf.reshape(1, OUT))
    return out.reshape(B, 12, L)


# speedup vs baseline: 16.4455x; 1.0769x over previous
import functools

import jax
import jax.numpy as jnp
from jax import lax
from jax.experimental import pallas as pl
from jax.experimental.pallas import tpu as pltpu
from jax.experimental.pallas import tpu_sc as plsc

B, T, L, H = 16, 128, 128, 128
N = B * T
E = 65536
OUT = 12 * L
NC, NS = 2, 16
NW = NC * NS
EPT = E // NW
CH = 128
NCH = EPT // CH
RPT = N // NS
NBUF = 4
DR = N // 16


def _remap16(v):
    return ((v & (T - 1)) << 4) | (v >> 7)


def _deg_body(edge_hbm, dp_hbm, er_hbm, srcv, dstv, degv, idxv, acc):
    cid = lax.axis_index("c")
    sid = lax.axis_index("s")
    wid = sid * NC + cid
    pltpu.sync_copy(edge_hbm.at[0, wid], srcv)
    pltpu.sync_copy(edge_hbm.at[1, wid], dstv)

    def zero(i, carry):
        def zcol(k, c2):
            degv[i, pl.ds(k * 16, 16)] = jnp.zeros((16,), jnp.float32)
            return c2
        return lax.fori_loop(0, H // 16, zcol, carry)
    lax.fori_loop(0, N // H, zero, 0)

    idxv[0, pl.ds(0, 16)] = lax.iota(jnp.int32, 16)

    pltpu.sync_copy(degv.at[pl.ds(0, 1)], acc.at[pl.ds(sid, 1)])

    ones = jnp.ones((16,), jnp.float32)

    def jloop(j, carry):
        def iloop(i, c2):
            vs = srcv[j, pl.ds(i * 16, 16)]
            srcv[j, pl.ds(i * 16, 16)] = _remap16(vs)
            vd = _remap16(dstv[j, pl.ds(i * 16, 16)])
            dstv[j, pl.ds(i * 16, 16)] = vd
            plsc.addupdate_scatter(degv, [vd >> 7, vd & (H - 1)], ones)
            return c2
        return lax.fori_loop(0, CH // 16, iloop, carry)
    lax.fori_loop(0, NCH, jloop, 0)

    pltpu.sync_copy(srcv, er_hbm.at[0, wid])
    pltpu.sync_copy(dstv, er_hbm.at[1, wid])
    plsc.subcore_barrier()
    pltpu.sync_copy(degv, acc.at[idxv.at[0]], add=True)
    plsc.subcore_barrier()
    pltpu.sync_copy(acc.at[pl.ds(sid, 1)], dp_hbm.at[cid, pl.ds(sid, 1)])


def _scatter_body(g_hbm, er_hbm, zero_hbm, part_hbm,
                  srcv, dstv, rows0, rows1, rows2, rows3, acc,
                  g0s, g1s, g2s, g3s, s0s, s1s, s2s, s3s):
    cid = lax.axis_index("c")
    sid = lax.axis_index("s")
    wid = sid * NC + cid
    pltpu.sync_copy(zero_hbm, acc.at[pl.ds(sid * RPT, RPT)])
    pltpu.sync_copy(er_hbm.at[0, wid], srcv)
    pltpu.sync_copy(er_hbm.at[1, wid], dstv)
    plsc.subcore_barrier()
    bufs = (rows0, rows1, rows2, rows3)
    gsems = (g0s, g1s, g2s, g3s)
    gdesc = [None] * NBUF
    gdesc[0] = pltpu.async_copy(g_hbm.at[srcv.at[0]], bufs[0], gsems[0])
    for j in range(NCH):
        b = j % 2
        gdesc[b].wait()
        if j + 1 < NCH:
            gdesc[1 - b] = pltpu.async_copy(
                g_hbm.at[srcv.at[j + 1]], bufs[1 - b], gsems[1 - b])
        pltpu.sync_copy(bufs[b], acc.at[dstv.at[j]], add=True)
    plsc.subcore_barrier()
    pltpu.sync_copy(acc.at[pl.ds(sid * RPT, RPT)],
                    part_hbm.at[cid, pl.ds(sid * RPT, RPT)])


def _proj_body(x_ref, d_ref, w0_ref, dp_ref, g0_ref, dinv_ref):
    c0 = jnp.dot(d_ref[...], w0_ref[L:, :],
                 preferred_element_type=jnp.float32)
    hw0 = jnp.dot(x_ref[...], w0_ref[:L, :],
                  preferred_element_type=jnp.float32) + c0
    deg = dp_ref[...] + 1.0
    dinv = lax.rsqrt(deg)
    dinv_ref[...] = dinv
    g0_ref[...] = hw0 * dinv


def _layer_body(s_ref, g_ref, dinv_ref, b_ref, w_ref, out_ref):
    dinv = dinv_ref[...]
    h = jnp.maximum(
        dinv * (s_ref[0] + s_ref[1] + g_ref[...]) + b_ref[...], 0.0)
    out_ref[...] = jnp.dot(
        h, w_ref[...], preferred_element_type=jnp.float32) * dinv


def _lstm_in_body(s_ref, g_ref, dinv_ref, b_ref, wih_ref,
                  bih_ref, bhh_ref, xg_ref):
    h2 = jnp.maximum(
        dinv_ref[...] * (s_ref[0] + s_ref[1] + g_ref[...]) + b_ref[...],
        0.0)
    xg_ref[...] = (jnp.dot(h2, wih_ref[...], preferred_element_type=jnp.float32)
                   + bih_ref[...] + bhh_ref[...])


def _lstm_body(xg_ref, whh_ref, wf_ref, bf_ref, out_ref):
    whh = whh_ref[...]

    def step(t, carry):
        hp, cp = carry
        gates = xg_ref[t] + jnp.dot(hp, whh,
                                    preferred_element_type=jnp.float32)
        i = jax.nn.sigmoid(gates[:, :H])
        f = jax.nn.sigmoid(gates[:, H:2 * H])
        g = jnp.tanh(gates[:, 2 * H:3 * H])
        o = jax.nn.sigmoid(gates[:, 3 * H:])
        c = f * cp + i * g
        hn = o * jnp.tanh(c)
        return (hn, c)

    init = (jnp.zeros((B, H), jnp.float32), jnp.zeros((B, H), jnp.float32))
    hn, _ = lax.fori_loop(0, T, step, init)
    out_ref[...] = jnp.dot(
        hn, wf_ref[...], preferred_element_type=jnp.float32) + bf_ref[...]


def _tc(body, out_shapes):
    return pl.pallas_call(body, out_shape=out_shapes)


@functools.cache
def _sc_kernels():
    mesh = plsc.VectorSubcoreMesh(
        core_axis_name="c", subcore_axis_name="s",
        num_cores=NC, num_subcores=NS)
    params = pltpu.CompilerParams(needs_layout_passes=False)
    deg = pl.kernel(
        _deg_body,
        out_type=(jax.ShapeDtypeStruct((NC, NS, H), jnp.float32),
                  jax.ShapeDtypeStruct((2, NW, NCH, CH), jnp.int32)),
        mesh=mesh,
        compiler_params=params,
        scratch_types=[
            pltpu.VMEM((NCH, CH), jnp.int32),
            pltpu.VMEM((NCH, CH), jnp.int32),
            pltpu.VMEM((NS, H), jnp.float32),
            pltpu.VMEM((1, 16), jnp.int32),
            pltpu.VMEM_SHARED((NS, H), jnp.float32),
        ],
    )
    scatter = pl.kernel(
        _scatter_body,
        out_type=jax.ShapeDtypeStruct((NC, N, H), jnp.float32),
        mesh=mesh,
        compiler_params=params,
        scratch_types=[
            pltpu.VMEM((NCH, CH), jnp.int32),
            pltpu.VMEM((NCH, CH), jnp.int32),
            pltpu.VMEM((CH, H), jnp.float32),
            pltpu.VMEM((CH, H), jnp.float32),
            pltpu.VMEM((CH, H), jnp.float32),
            pltpu.VMEM((CH, H), jnp.float32),
            pltpu.VMEM_SHARED((N, H), jnp.float32),
        ] + [pltpu.SemaphoreType.DMA] * (2 * NBUF),
    )
    return deg, scatter


def kernel(x, distance, edge_index, W0, b0, W1, b1, Wih, Whh, bih, bhh, Wf, bf):
    x2d = x.transpose(1, 0, 2).reshape(N, L)
    dflat = distance.reshape(1, L * L)
    edge_r = edge_index.reshape(2, NW, NCH, CH)
    zero_rows = jnp.zeros((RPT, H), jnp.float32)
    _deg_kernel, _scatter_kernel = _sc_kernels()

    dp, er = _deg_kernel(edge_r)
    dpcol = (dp[0] + dp[1]).reshape(N, 1)

    g0, dinv = _tc(_proj_body, [
        jax.ShapeDtypeStruct((N, H), jnp.float32),
        jax.ShapeDtypeStruct((N, 1), jnp.float32),
    ])(x2d, dflat, W0, dpcol)

    s0 = _scatter_kernel(g0, er, zero_rows)
    g1 = _tc(_layer_body, jax.ShapeDtypeStruct((N, H), jnp.float32))(
        s0, g0, dinv, b0.reshape(1, H), W1)

    s1 = _scatter_kernel(g1, er, zero_rows)
    xg = _tc(_lstm_in_body, jax.ShapeDtypeStruct((N, 4 * H), jnp.float32))(
        s1, g1, dinv, b1.reshape(1, H), Wih.T,
        bih.reshape(1, 4 * H), bhh.reshape(1, 4 * H))

    xg3 = xg.reshape(T, B, 4 * H)
    out = _tc(_lstm_body, jax.ShapeDtypeStruct((B, OUT), jnp.float32))(
        xg3, Whh.T, Wf, bf.reshape(1, OUT))
    return out.reshape(B, 12, L)


# speedup vs baseline: 17.4212x; 1.0593x over previous
import functools

import jax
import jax.numpy as jnp
from jax import lax
from jax.experimental import pallas as pl
from jax.experimental.pallas import tpu as pltpu
from jax.experimental.pallas import tpu_sc as plsc

B, T, L, H = 16, 128, 128, 128
N = B * T
E = 65536
OUT = 12 * L
NC, NS = 2, 16
NW = NC * NS
EPT = E // NW
CH = 128
NCH = EPT // CH
RPT = N // NS
NBUF = 4
DR = N // 16


def _remap16(v):
    return ((v & (T - 1)) << 4) | (v >> 7)


def _deg_body(edge_hbm, dp_hbm, er_hbm, srcv, dstv, degv, idxv, acc):
    cid = lax.axis_index("c")
    sid = lax.axis_index("s")
    wid = sid * NC + cid
    pltpu.sync_copy(edge_hbm.at[0, wid], srcv)
    pltpu.sync_copy(edge_hbm.at[1, wid], dstv)

    def zero(i, carry):
        def zcol(k, c2):
            degv[i, pl.ds(k * 16, 16)] = jnp.zeros((16,), jnp.float32)
            return c2
        return lax.fori_loop(0, H // 16, zcol, carry)
    lax.fori_loop(0, N // H, zero, 0)

    idxv[0, pl.ds(0, 16)] = lax.iota(jnp.int32, 16)

    pltpu.sync_copy(degv.at[pl.ds(0, 1)], acc.at[pl.ds(sid, 1)])

    ones = jnp.ones((16,), jnp.float32)

    def jloop(j, carry):
        def iloop(i, c2):
            vs = srcv[j, pl.ds(i * 16, 16)]
            srcv[j, pl.ds(i * 16, 16)] = _remap16(vs)
            vd = _remap16(dstv[j, pl.ds(i * 16, 16)])
            dstv[j, pl.ds(i * 16, 16)] = vd
            plsc.addupdate_scatter(degv, [vd >> 7, vd & (H - 1)], ones)
            return c2
        return lax.fori_loop(0, CH // 16, iloop, carry)
    lax.fori_loop(0, NCH, jloop, 0)

    pltpu.sync_copy(srcv, er_hbm.at[0, wid])
    pltpu.sync_copy(dstv, er_hbm.at[1, wid])
    plsc.subcore_barrier()
    pltpu.sync_copy(degv, acc.at[idxv.at[0]], add=True)
    plsc.subcore_barrier()
    pltpu.sync_copy(acc.at[pl.ds(sid, 1)], dp_hbm.at[cid, pl.ds(sid, 1)])


def _scatter_body(g_hbm, er_hbm, zero_hbm, part_hbm,
                  srcv, dstv, rows0, rows1, rows2, rows3, acc,
                  g0s, g1s, g2s, g3s, s0s, s1s, s2s, s3s):
    cid = lax.axis_index("c")
    sid = lax.axis_index("s")
    wid = sid * NC + cid
    pltpu.sync_copy(zero_hbm, acc.at[pl.ds(sid * RPT, RPT)])
    pltpu.sync_copy(er_hbm.at[0, wid], srcv)
    pltpu.sync_copy(er_hbm.at[1, wid], dstv)
    plsc.subcore_barrier()
    bufs = (rows0, rows1, rows2, rows3)
    gsems = (g0s, g1s, g2s, g3s)
    ssems = (s0s, s1s, s2s, s3s)
    gdesc = [None] * NBUF
    sdesc = [None] * NBUF
    for k in range(2):
        gdesc[k] = pltpu.async_copy(g_hbm.at[srcv.at[k]], bufs[k], gsems[k])
    for j in range(NCH):
        b = j % NBUF
        gdesc[b].wait()
        sdesc[b] = pltpu.async_copy(
            bufs[b], acc.at[dstv.at[j]], ssems[b], add=True)
        nj = j + 2
        if nj < NCH:
            nb = nj % NBUF
            if sdesc[nb] is not None:
                sdesc[nb].wait()
            gdesc[nb] = pltpu.async_copy(
                g_hbm.at[srcv.at[nj]], bufs[nb], gsems[nb])
    for j in range(NCH - NBUF, NCH):
        sdesc[j % NBUF].wait()
    plsc.subcore_barrier()
    pltpu.sync_copy(acc.at[pl.ds(sid * RPT, RPT)],
                    part_hbm.at[cid, pl.ds(sid * RPT, RPT)])


def _proj_body(x_ref, d_ref, w0_ref, dp_ref, g0_ref, dinv_ref):
    c0 = jnp.dot(d_ref[...], w0_ref[L:, :],
                 preferred_element_type=jnp.float32)
    hw0 = jnp.dot(x_ref[...], w0_ref[:L, :],
                  preferred_element_type=jnp.float32) + c0
    deg = dp_ref[...] + 1.0
    dinv = lax.rsqrt(deg)
    dinv_ref[...] = dinv
    g0_ref[...] = hw0 * dinv


def _layer_body(s_ref, g_ref, dinv_ref, b_ref, w_ref, out_ref):
    dinv = dinv_ref[...]
    h = jnp.maximum(
        dinv * (s_ref[0] + s_ref[1] + g_ref[...]) + b_ref[...], 0.0)
    out_ref[...] = jnp.dot(
        h, w_ref[...], preferred_element_type=jnp.float32) * dinv


def _lstm_in_body(s_ref, g_ref, dinv_ref, b_ref, wih_ref,
                  bih_ref, bhh_ref, xg_ref):
    h2 = jnp.maximum(
        dinv_ref[...] * (s_ref[0] + s_ref[1] + g_ref[...]) + b_ref[...],
        0.0)
    xg_ref[...] = (jnp.dot(h2, wih_ref[...], preferred_element_type=jnp.float32)
                   + bih_ref[...] + bhh_ref[...])


def _lstm_body(xg_ref, whh_ref, wf_ref, bf_ref, out_ref):
    whh = whh_ref[...]

    def step(t, carry):
        hp, cp = carry
        gates = xg_ref[t] + jnp.dot(hp, whh,
                                    preferred_element_type=jnp.float32)
        i = jax.nn.sigmoid(gates[:, :H])
        f = jax.nn.sigmoid(gates[:, H:2 * H])
        g = jnp.tanh(gates[:, 2 * H:3 * H])
        o = jax.nn.sigmoid(gates[:, 3 * H:])
        c = f * cp + i * g
        hn = o * jnp.tanh(c)
        return (hn, c)

    init = (jnp.zeros((B, H), jnp.float32), jnp.zeros((B, H), jnp.float32))
    hn, _ = lax.fori_loop(0, T, step, init)
    out_ref[...] = jnp.dot(
        hn, wf_ref[...], preferred_element_type=jnp.float32) + bf_ref[...]


def _tc(body, out_shapes):
    return pl.pallas_call(body, out_shape=out_shapes)


@functools.cache
def _sc_kernels():
    mesh = plsc.VectorSubcoreMesh(
        core_axis_name="c", subcore_axis_name="s",
        num_cores=NC, num_subcores=NS)
    params = pltpu.CompilerParams(needs_layout_passes=False)
    deg = pl.kernel(
        _deg_body,
        out_type=(jax.ShapeDtypeStruct((NC, NS, H), jnp.float32),
                  jax.ShapeDtypeStruct((2, NW, NCH, CH), jnp.int32)),
        mesh=mesh,
        compiler_params=params,
        scratch_types=[
            pltpu.VMEM((NCH, CH), jnp.int32),
            pltpu.VMEM((NCH, CH), jnp.int32),
            pltpu.VMEM((NS, H), jnp.float32),
            pltpu.VMEM((1, 16), jnp.int32),
            pltpu.VMEM_SHARED((NS, H), jnp.float32),
        ],
    )
    scatter = pl.kernel(
        _scatter_body,
        out_type=jax.ShapeDtypeStruct((NC, N, H), jnp.float32),
        mesh=mesh,
        compiler_params=params,
        scratch_types=[
            pltpu.VMEM((NCH, CH), jnp.int32),
            pltpu.VMEM((NCH, CH), jnp.int32),
            pltpu.VMEM((CH, H), jnp.float32),
            pltpu.VMEM((CH, H), jnp.float32),
            pltpu.VMEM((CH, H), jnp.float32),
            pltpu.VMEM((CH, H), jnp.float32),
            pltpu.VMEM_SHARED((N, H), jnp.float32),
        ] + [pltpu.SemaphoreType.DMA] * (2 * NBUF),
    )
    return deg, scatter


def kernel(x, distance, edge_index, W0, b0, W1, b1, Wih, Whh, bih, bhh, Wf, bf):
    x2d = x.transpose(1, 0, 2).reshape(N, L)
    dflat = distance.reshape(1, L * L)
    edge_r = edge_index.reshape(2, NW, NCH, CH)
    zero_rows = jnp.zeros((RPT, H), jnp.float32)
    _deg_kernel, _scatter_kernel = _sc_kernels()

    dp, er = _deg_kernel(edge_r)
    dpcol = (dp[0] + dp[1]).reshape(N, 1)

    g0, dinv = _tc(_proj_body, [
        jax.ShapeDtypeStruct((N, H), jnp.float32),
        jax.ShapeDtypeStruct((N, 1), jnp.float32),
    ])(x2d, dflat, W0, dpcol)

    s0 = _scatter_kernel(g0, er, zero_rows)
    g1 = _tc(_layer_body, jax.ShapeDtypeStruct((N, H), jnp.float32))(
        s0, g0, dinv, b0.reshape(1, H), W1)

    s1 = _scatter_kernel(g1, er, zero_rows)
    xg = _tc(_lstm_in_body, jax.ShapeDtypeStruct((N, 4 * H), jnp.float32))(
        s1, g1, dinv, b1.reshape(1, H), Wih.T,
        bih.reshape(1, 4 * H), bhh.reshape(1, 4 * H))

    xg3 = xg.reshape(T, B, 4 * H)
    out = _tc(_lstm_body, jax.ShapeDtypeStruct((B, OUT), jnp.float32))(
        xg3, Whh.T, Wf, bf.reshape(1, OUT))
    return out.reshape(B, 12, L)


# speedup vs baseline: 18.5866x; 1.0669x over previous
import functools

import jax
import jax.numpy as jnp
from jax import lax
from jax.experimental import pallas as pl
from jax.experimental.pallas import tpu as pltpu
from jax.experimental.pallas import tpu_sc as plsc

B, T, L, H = 16, 128, 128, 128
N = B * T
E = 65536
OUT = 12 * L
NC, NS = 2, 16
NW = NC * NS
EPT = E // NW
CH = 128
NCH = EPT // CH
RPT = N // NS
NBUF = 4
DR = N // 16


def _remap16(v):
    return ((v & (T - 1)) << 4) | (v >> 7)


def _deg_body(edge_hbm, dp_hbm, er_hbm, srcv, dstv, degv, idxv, acc):
    cid = lax.axis_index("c")
    sid = lax.axis_index("s")
    wid = sid * NC + cid
    pltpu.sync_copy(edge_hbm.at[0, wid], srcv)
    pltpu.sync_copy(edge_hbm.at[1, wid], dstv)

    def zero(i, carry):
        def zcol(k, c2):
            degv[i, pl.ds(k * 16, 16)] = jnp.zeros((16,), jnp.float32)
            return c2
        return lax.fori_loop(0, H // 16, zcol, carry)
    lax.fori_loop(0, N // H, zero, 0)

    idxv[0, pl.ds(0, 16)] = lax.iota(jnp.int32, 16)

    pltpu.sync_copy(degv.at[pl.ds(0, 1)], acc.at[pl.ds(sid, 1)])

    ones = jnp.ones((16,), jnp.float32)

    def jloop(j, carry):
        def iloop(i, c2):
            vs = srcv[j, pl.ds(i * 16, 16)]
            srcv[j, pl.ds(i * 16, 16)] = _remap16(vs)
            vd = _remap16(dstv[j, pl.ds(i * 16, 16)])
            dstv[j, pl.ds(i * 16, 16)] = vd
            plsc.addupdate_scatter(degv, [vd >> 7, vd & (H - 1)], ones)
            return c2
        return lax.fori_loop(0, CH // 16, iloop, carry)
    lax.fori_loop(0, NCH, jloop, 0)

    pltpu.sync_copy(srcv, er_hbm.at[0, wid])
    pltpu.sync_copy(dstv, er_hbm.at[1, wid])
    plsc.subcore_barrier()
    pltpu.sync_copy(degv, acc.at[idxv.at[0]], add=True)
    plsc.subcore_barrier()
    pltpu.sync_copy(acc.at[pl.ds(sid, 1)], dp_hbm.at[cid, pl.ds(sid, 1)])


def _scatter_body(g_hbm, er_hbm, zero_hbm, part_hbm,
                  srcv, dstv, rows0, rows1, rows2, rows3, acc,
                  g0s, g1s, g2s, g3s, s0s, s1s, s2s, s3s):
    cid = lax.axis_index("c")
    sid = lax.axis_index("s")
    wid = sid * NC + cid
    pltpu.sync_copy(er_hbm.at[0, wid], srcv)
    pltpu.sync_copy(er_hbm.at[1, wid], dstv)
    bufs = (rows0, rows1, rows2, rows3)
    gsems = (g0s, g1s, g2s, g3s)
    ssems = (s0s, s1s, s2s, s3s)
    gdesc = [None] * NBUF
    sdesc = [None] * NBUF
    for k in range(2):
        gdesc[k] = pltpu.async_copy(g_hbm.at[srcv.at[k]], bufs[k], gsems[k])
    pltpu.sync_copy(zero_hbm, acc.at[pl.ds(sid * RPT, RPT)])
    plsc.subcore_barrier()
    for j in range(NCH):
        b = j % NBUF
        gdesc[b].wait()
        sdesc[b] = pltpu.async_copy(
            bufs[b], acc.at[dstv.at[j]], ssems[b], add=True)
        nj = j + 2
        if nj < NCH:
            nb = nj % NBUF
            if sdesc[nb] is not None:
                sdesc[nb].wait()
            gdesc[nb] = pltpu.async_copy(
                g_hbm.at[srcv.at[nj]], bufs[nb], gsems[nb])
    for j in range(NCH - NBUF, NCH):
        sdesc[j % NBUF].wait()
    plsc.subcore_barrier()
    pltpu.sync_copy(acc.at[pl.ds(sid * RPT, RPT)],
                    part_hbm.at[cid, pl.ds(sid * RPT, RPT)])


def _hw0_body(x_ref, d_ref, w0_ref, hw0_ref):
    c0 = jnp.dot(d_ref[...], w0_ref[L:, :],
                 preferred_element_type=jnp.float32)
    hw0_ref[...] = jnp.dot(x_ref[...], w0_ref[:L, :],
                           preferred_element_type=jnp.float32) + c0


def _scale_body(hw0_ref, dp_ref, g0_ref, dinv_ref):
    deg = dp_ref[...] + 1.0
    dinv = lax.rsqrt(deg)
    dinv_ref[...] = dinv
    g0_ref[...] = hw0_ref[...] * dinv


def _layer_body(s_ref, g_ref, dinv_ref, b_ref, w_ref, out_ref):
    dinv = dinv_ref[...]
    h = jnp.maximum(
        dinv * (s_ref[0] + s_ref[1] + g_ref[...]) + b_ref[...], 0.0)
    out_ref[...] = jnp.dot(
        h, w_ref[...], preferred_element_type=jnp.float32) * dinv


def _lstm_body(s_ref, g_ref, dinv_ref, b_ref, wih_ref, bih_ref, bhh_ref,
               whh_ref, wf_ref, bf_ref, out_ref, xg_scr):
    h2 = jnp.maximum(
        dinv_ref[...] * (s_ref[0] + s_ref[1] + g_ref[...]) + b_ref[...],
        0.0)
    xg_scr[...] = (jnp.dot(h2, wih_ref[...], preferred_element_type=jnp.float32)
                   + bih_ref[...] + bhh_ref[...])
    whh = whh_ref[...]

    def step(t, carry):
        hp, cp = carry
        xt = xg_scr[pl.ds(t * B, B), :]
        gates = xt + jnp.dot(hp, whh, preferred_element_type=jnp.float32)
        i = jax.nn.sigmoid(gates[:, :H])
        f = jax.nn.sigmoid(gates[:, H:2 * H])
        g = jnp.tanh(gates[:, 2 * H:3 * H])
        o = jax.nn.sigmoid(gates[:, 3 * H:])
        c = f * cp + i * g
        hn = o * jnp.tanh(c)
        return (hn, c)

    init = (jnp.zeros((B, H), jnp.float32), jnp.zeros((B, H), jnp.float32))
    hn, _ = lax.fori_loop(0, T, step, init)
    out_ref[...] = jnp.dot(
        hn, wf_ref[...], preferred_element_type=jnp.float32) + bf_ref[...]


def _tc(body, out_shapes):
    return pl.pallas_call(body, out_shape=out_shapes)


@functools.cache
def _sc_kernels():
    mesh = plsc.VectorSubcoreMesh(
        core_axis_name="c", subcore_axis_name="s",
        num_cores=NC, num_subcores=NS)
    params = pltpu.CompilerParams(needs_layout_passes=False)
    deg = pl.kernel(
        _deg_body,
        out_type=(jax.ShapeDtypeStruct((NC, NS, H), jnp.float32),
                  jax.ShapeDtypeStruct((2, NW, NCH, CH), jnp.int32)),
        mesh=mesh,
        compiler_params=params,
        scratch_types=[
            pltpu.VMEM((NCH, CH), jnp.int32),
            pltpu.VMEM((NCH, CH), jnp.int32),
            pltpu.VMEM((NS, H), jnp.float32),
            pltpu.VMEM((1, 16), jnp.int32),
            pltpu.VMEM_SHARED((NS, H), jnp.float32),
        ],
    )
    scatter = pl.kernel(
        _scatter_body,
        out_type=jax.ShapeDtypeStruct((NC, N, H), jnp.float32),
        mesh=mesh,
        compiler_params=params,
        scratch_types=[
            pltpu.VMEM((NCH, CH), jnp.int32),
            pltpu.VMEM((NCH, CH), jnp.int32),
            pltpu.VMEM((CH, H), jnp.float32),
            pltpu.VMEM((CH, H), jnp.float32),
            pltpu.VMEM((CH, H), jnp.float32),
            pltpu.VMEM((CH, H), jnp.float32),
            pltpu.VMEM_SHARED((N, H), jnp.float32),
        ] + [pltpu.SemaphoreType.DMA] * (2 * NBUF),
    )
    return deg, scatter


def kernel(x, distance, edge_index, W0, b0, W1, b1, Wih, Whh, bih, bhh, Wf, bf):
    x2d = x.transpose(1, 0, 2).reshape(N, L)
    dflat = distance.reshape(1, L * L)
    edge_r = edge_index.reshape(2, NW, NCH, CH)
    zero_rows = jnp.zeros((RPT, H), jnp.float32)
    _deg_kernel, _scatter_kernel = _sc_kernels()

    dp, er = _deg_kernel(edge_r)
    dpcol = (dp[0] + dp[1]).reshape(N, 1)

    hw0 = _tc(_hw0_body, jax.ShapeDtypeStruct((N, H), jnp.float32))(
        x2d, dflat, W0)
    g0, dinv = _tc(_scale_body, [
        jax.ShapeDtypeStruct((N, H), jnp.float32),
        jax.ShapeDtypeStruct((N, 1), jnp.float32),
    ])(hw0, dpcol)

    s0 = _scatter_kernel(g0, er, zero_rows)
    g1 = _tc(_layer_body, jax.ShapeDtypeStruct((N, H), jnp.float32))(
        s0, g0, dinv, b0.reshape(1, H), W1)

    s1 = _scatter_kernel(g1, er, zero_rows)
    out = pl.pallas_call(
        _lstm_body,
        out_shape=jax.ShapeDtypeStruct((B, OUT), jnp.float32),
        scratch_shapes=[pltpu.VMEM((N, 4 * H), jnp.float32)],
    )(s1, g1, dinv, b1.reshape(1, H), Wih.T,
      bih.reshape(1, 4 * H), bhh.reshape(1, 4 * H),
      Whh.T, Wf, bf.reshape(1, OUT))
    return out.reshape(B, 12, L)


# speedup vs baseline: 18.7229x; 1.0073x over previous
import functools

import jax
import jax.numpy as jnp
from jax import lax
from jax.experimental import pallas as pl
from jax.experimental.pallas import tpu as pltpu
from jax.experimental.pallas import tpu_sc as plsc

B, T, L, H = 16, 128, 128, 128
N = B * T
E = 65536
OUT = 12 * L
NC, NS = 2, 16
NW = NC * NS
EPT = E // NW
CH = 128
NCH = EPT // CH
RPT = N // NS
NBUF = 4
DR = N // 16


def _remap16(v):
    return ((v & (T - 1)) << 4) | (v >> 7)


def _deg_body(edge_hbm, dp_hbm, er_hbm, srcv, dstv, degv, idxv, acc):
    cid = lax.axis_index("c")
    sid = lax.axis_index("s")
    wid = sid * NC + cid
    pltpu.sync_copy(edge_hbm.at[0, wid], srcv)
    pltpu.sync_copy(edge_hbm.at[1, wid], dstv)

    def zero(i, carry):
        def zcol(k, c2):
            degv[i, pl.ds(k * 16, 16)] = jnp.zeros((16,), jnp.float32)
            return c2
        return lax.fori_loop(0, H // 16, zcol, carry)
    lax.fori_loop(0, N // H, zero, 0)

    idxv[0, pl.ds(0, 16)] = lax.iota(jnp.int32, 16)

    pltpu.sync_copy(degv.at[pl.ds(0, 1)], acc.at[pl.ds(sid, 1)])

    ones = jnp.ones((16,), jnp.float32)

    def jloop(j, carry):
        def iloop(i, c2):
            vs = srcv[j, pl.ds(i * 16, 16)]
            srcv[j, pl.ds(i * 16, 16)] = _remap16(vs)
            vd = _remap16(dstv[j, pl.ds(i * 16, 16)])
            dstv[j, pl.ds(i * 16, 16)] = vd
            plsc.addupdate_scatter(degv, [vd >> 7, vd & (H - 1)], ones)
            return c2
        return lax.fori_loop(0, CH // 16, iloop, carry)
    lax.fori_loop(0, NCH, jloop, 0)

    pltpu.sync_copy(srcv, er_hbm.at[0, wid])
    pltpu.sync_copy(dstv, er_hbm.at[1, wid])
    plsc.subcore_barrier()
    pltpu.sync_copy(degv, acc.at[idxv.at[0]], add=True)
    plsc.subcore_barrier()
    pltpu.sync_copy(acc.at[pl.ds(sid, 1)], dp_hbm.at[cid, pl.ds(sid, 1)])


def _scatter_body(g_hbm, er_hbm, zero_hbm, part_hbm,
                  srcv, dstv, rows0, rows1, rows2, rows3, acc,
                  g0s, g1s, g2s, g3s, s0s, s1s, s2s, s3s):
    cid = lax.axis_index("c")
    sid = lax.axis_index("s")
    wid = sid * NC + cid
    pltpu.sync_copy(er_hbm.at[0, wid], srcv)
    pltpu.sync_copy(er_hbm.at[1, wid], dstv)
    bufs = (rows0, rows1, rows2, rows3)
    gsems = (g0s, g1s, g2s, g3s)
    ssems = (s0s, s1s, s2s, s3s)
    gdesc = [None] * NBUF
    sdesc = [None] * NBUF
    for k in range(2):
        gdesc[k] = pltpu.async_copy(g_hbm.at[srcv.at[k]], bufs[k], gsems[k])
    pltpu.sync_copy(zero_hbm, acc.at[pl.ds(sid * RPT, RPT)])
    plsc.subcore_barrier()
    for j in range(NCH):
        b = j % NBUF
        gdesc[b].wait()
        sdesc[b] = pltpu.async_copy(
            bufs[b], acc.at[dstv.at[j]], ssems[b], add=True)
        nj = j + 2
        if nj < NCH:
            nb = nj % NBUF
            if sdesc[nb] is not None:
                sdesc[nb].wait()
            gdesc[nb] = pltpu.async_copy(
                g_hbm.at[srcv.at[nj]], bufs[nb], gsems[nb])
    for j in range(NCH - NBUF, NCH):
        sdesc[j % NBUF].wait()
    plsc.subcore_barrier()
    pltpu.sync_copy(acc.at[pl.ds(sid * RPT, RPT)],
                    part_hbm.at[cid, pl.ds(sid * RPT, RPT)])


def _hw0_body(x_ref, d_ref, w0_ref, hw0_ref):
    c0 = jnp.dot(d_ref[...], w0_ref[L:, :],
                 preferred_element_type=jnp.float32)
    hw0_ref[...] = jnp.dot(x_ref[...], w0_ref[:L, :],
                           preferred_element_type=jnp.float32) + c0


def _scale_body(hw0_ref, dp_ref, g0_ref, dinv_ref):
    deg = dp_ref[...] + 1.0
    dinv = lax.rsqrt(deg)
    dinv_ref[...] = dinv
    g0_ref[...] = hw0_ref[...] * dinv


def _layer_body(s_ref, g_ref, dinv_ref, b_ref, w_ref, out_ref):
    dinv = dinv_ref[...]
    h = jnp.maximum(
        dinv * (s_ref[0] + s_ref[1] + g_ref[...]) + b_ref[...], 0.0)
    out_ref[...] = jnp.dot(
        h, w_ref[...], preferred_element_type=jnp.float32) * dinv


def _lstm_body(s_ref, g_ref, dinv_ref, b_ref, wih_ref, bih_ref, bhh_ref,
               whh_ref, wf_ref, bf_ref, out_ref, xg_scr):
    h2 = jnp.maximum(
        dinv_ref[...] * (s_ref[0] + s_ref[1] + g_ref[...]) + b_ref[...],
        0.0)
    xg_scr[...] = (jnp.dot(h2, wih_ref[...], preferred_element_type=jnp.float32)
                   + bih_ref[...] + bhh_ref[...])
    whh = whh_ref[...]

    def cell(t, hp, cp):
        xt = xg_scr[pl.ds(t * B, B), :]
        gates = xt + jnp.dot(hp, whh, preferred_element_type=jnp.float32)
        i = jax.nn.sigmoid(gates[:, :H])
        f = jax.nn.sigmoid(gates[:, H:2 * H])
        g = jnp.tanh(gates[:, 2 * H:3 * H])
        o = jax.nn.sigmoid(gates[:, 3 * H:])
        c = f * cp + i * g
        hn = o * jnp.tanh(c)
        return hn, c

    def step(k, carry):
        hp, cp = carry
        hp, cp = cell(2 * k, hp, cp)
        hp, cp = cell(2 * k + 1, hp, cp)
        return (hp, cp)

    init = (jnp.zeros((B, H), jnp.float32), jnp.zeros((B, H), jnp.float32))
    hn, _ = lax.fori_loop(0, T // 2, step, init)
    out_ref[...] = jnp.dot(
        hn, wf_ref[...], preferred_element_type=jnp.float32) + bf_ref[...]


def _tc(body, out_shapes):
    return pl.pallas_call(body, out_shape=out_shapes)


@functools.cache
def _sc_kernels():
    mesh = plsc.VectorSubcoreMesh(
        core_axis_name="c", subcore_axis_name="s",
        num_cores=NC, num_subcores=NS)
    params = pltpu.CompilerParams(needs_layout_passes=False)
    deg = pl.kernel(
        _deg_body,
        out_type=(jax.ShapeDtypeStruct((NC, NS, H), jnp.float32),
                  jax.ShapeDtypeStruct((2, NW, NCH, CH), jnp.int32)),
        mesh=mesh,
        compiler_params=params,
        scratch_types=[
            pltpu.VMEM((NCH, CH), jnp.int32),
            pltpu.VMEM((NCH, CH), jnp.int32),
            pltpu.VMEM((NS, H), jnp.float32),
            pltpu.VMEM((1, 16), jnp.int32),
            pltpu.VMEM_SHARED((NS, H), jnp.float32),
        ],
    )
    scatter = pl.kernel(
        _scatter_body,
        out_type=jax.ShapeDtypeStruct((NC, N, H), jnp.float32),
        mesh=mesh,
        compiler_params=params,
        scratch_types=[
            pltpu.VMEM((NCH, CH), jnp.int32),
            pltpu.VMEM((NCH, CH), jnp.int32),
            pltpu.VMEM((CH, H), jnp.float32),
            pltpu.VMEM((CH, H), jnp.float32),
            pltpu.VMEM((CH, H), jnp.float32),
            pltpu.VMEM((CH, H), jnp.float32),
            pltpu.VMEM_SHARED((N, H), jnp.float32),
        ] + [pltpu.SemaphoreType.DMA] * (2 * NBUF),
    )
    return deg, scatter


def kernel(x, distance, edge_index, W0, b0, W1, b1, Wih, Whh, bih, bhh, Wf, bf):
    x2d = x.transpose(1, 0, 2).reshape(N, L)
    dflat = distance.reshape(1, L * L)
    edge_r = edge_index.reshape(2, NW, NCH, CH)
    zero_rows = jnp.zeros((RPT, H), jnp.float32)
    _deg_kernel, _scatter_kernel = _sc_kernels()

    dp, er = _deg_kernel(edge_r)
    dpcol = (dp[0] + dp[1]).reshape(N, 1)

    hw0 = _tc(_hw0_body, jax.ShapeDtypeStruct((N, H), jnp.float32))(
        x2d, dflat, W0)
    g0, dinv = _tc(_scale_body, [
        jax.ShapeDtypeStruct((N, H), jnp.float32),
        jax.ShapeDtypeStruct((N, 1), jnp.float32),
    ])(hw0, dpcol)

    s0 = _scatter_kernel(g0, er, zero_rows)
    g1 = _tc(_layer_body, jax.ShapeDtypeStruct((N, H), jnp.float32))(
        s0, g0, dinv, b0.reshape(1, H), W1)

    s1 = _scatter_kernel(g1, er, zero_rows)
    out = pl.pallas_call(
        _lstm_body,
        out_shape=jax.ShapeDtypeStruct((B, OUT), jnp.float32),
        scratch_shapes=[pltpu.VMEM((N, 4 * H), jnp.float32)],
    )(s1, g1, dinv, b1.reshape(1, H), Wih.T,
      bih.reshape(1, 4 * H), bhh.reshape(1, 4 * H),
      Whh.T, Wf, bf.reshape(1, OUT))
    return out.reshape(B, 12, L)


# speedup vs baseline: 18.8038x; 1.0043x over previous
import functools

import jax
import jax.numpy as jnp
from jax import lax
from jax.experimental import pallas as pl
from jax.experimental.pallas import tpu as pltpu
from jax.experimental.pallas import tpu_sc as plsc

B, T, L, H = 16, 128, 128, 128
N = B * T
E = 65536
OUT = 12 * L
NC, NS = 2, 16
NW = NC * NS
EPT = E // NW
CH = 128
NCH = EPT // CH
RPT = N // NS
NBUF = 6
DR = N // 16


def _remap16(v):
    return ((v & (T - 1)) << 4) | (v >> 7)


def _deg_body(edge_hbm, dp_hbm, er_hbm, srcv, dstv, degv, idxv, acc):
    cid = lax.axis_index("c")
    sid = lax.axis_index("s")
    wid = sid * NC + cid
    pltpu.sync_copy(edge_hbm.at[0, wid], srcv)
    pltpu.sync_copy(edge_hbm.at[1, wid], dstv)

    def zero(i, carry):
        def zcol(k, c2):
            degv[i, pl.ds(k * 16, 16)] = jnp.zeros((16,), jnp.float32)
            return c2
        return lax.fori_loop(0, H // 16, zcol, carry)
    lax.fori_loop(0, N // H, zero, 0)

    idxv[0, pl.ds(0, 16)] = lax.iota(jnp.int32, 16)

    pltpu.sync_copy(degv.at[pl.ds(0, 1)], acc.at[pl.ds(sid, 1)])

    ones = jnp.ones((16,), jnp.float32)

    def jloop(j, carry):
        def iloop(i, c2):
            vs = srcv[j, pl.ds(i * 16, 16)]
            srcv[j, pl.ds(i * 16, 16)] = _remap16(vs)
            vd = _remap16(dstv[j, pl.ds(i * 16, 16)])
            dstv[j, pl.ds(i * 16, 16)] = vd
            plsc.addupdate_scatter(degv, [vd >> 7, vd & (H - 1)], ones)
            return c2
        return lax.fori_loop(0, CH // 16, iloop, carry)
    lax.fori_loop(0, NCH, jloop, 0)

    pltpu.sync_copy(srcv, er_hbm.at[0, wid])
    pltpu.sync_copy(dstv, er_hbm.at[1, wid])
    plsc.subcore_barrier()
    pltpu.sync_copy(degv, acc.at[idxv.at[0]], add=True)
    plsc.subcore_barrier()
    pltpu.sync_copy(acc.at[pl.ds(sid, 1)], dp_hbm.at[cid, pl.ds(sid, 1)])


def _scatter_body(g_hbm, er_hbm, zero_hbm, part_hbm,
                  srcv, dstv, rows0, rows1, rows2, rows3, rows4, rows5, acc,
                  g0s, g1s, g2s, g3s, g4s, g5s,
                  s0s, s1s, s2s, s3s, s4s, s5s):
    cid = lax.axis_index("c")
    sid = lax.axis_index("s")
    wid = sid * NC + cid
    pltpu.sync_copy(er_hbm.at[0, wid], srcv)
    pltpu.sync_copy(er_hbm.at[1, wid], dstv)
    bufs = (rows0, rows1, rows2, rows3, rows4, rows5)
    gsems = (g0s, g1s, g2s, g3s, g4s, g5s)
    ssems = (s0s, s1s, s2s, s3s, s4s, s5s)
    gdesc = [None] * NBUF
    sdesc = [None] * NBUF
    for k in range(2):
        gdesc[k] = pltpu.async_copy(g_hbm.at[srcv.at[k]], bufs[k], gsems[k])
    pltpu.sync_copy(zero_hbm, acc.at[pl.ds(sid * RPT, RPT)])
    plsc.subcore_barrier()
    for j in range(NCH):
        b = j % NBUF
        gdesc[b].wait()
        sdesc[b] = pltpu.async_copy(
            bufs[b], acc.at[dstv.at[j]], ssems[b], add=True)
        nj = j + 2
        if nj < NCH:
            nb = nj % NBUF
            if sdesc[nb] is not None:
                sdesc[nb].wait()
            gdesc[nb] = pltpu.async_copy(
                g_hbm.at[srcv.at[nj]], bufs[nb], gsems[nb])
    for j in range(NCH - NBUF, NCH):
        sdesc[j % NBUF].wait()
    plsc.subcore_barrier()
    pltpu.sync_copy(acc.at[pl.ds(sid * RPT, RPT)],
                    part_hbm.at[cid, pl.ds(sid * RPT, RPT)])


def _hw0_body(x_ref, d_ref, w0_ref, hw0_ref):
    c0 = jnp.dot(d_ref[...], w0_ref[L:, :],
                 preferred_element_type=jnp.float32)
    hw0_ref[...] = jnp.dot(x_ref[...], w0_ref[:L, :],
                           preferred_element_type=jnp.float32) + c0


def _scale_body(hw0_ref, dp_ref, g0_ref, dinv_ref):
    deg = dp_ref[...] + 1.0
    dinv = lax.rsqrt(deg)
    dinv_ref[...] = dinv
    g0_ref[...] = hw0_ref[...] * dinv


def _layer_body(s_ref, g_ref, dinv_ref, b_ref, w_ref, out_ref):
    dinv = dinv_ref[...]
    h = jnp.maximum(
        dinv * (s_ref[0] + s_ref[1] + g_ref[...]) + b_ref[...], 0.0)
    out_ref[...] = jnp.dot(
        h, w_ref[...], preferred_element_type=jnp.float32) * dinv


def _lstm_body(s_ref, g_ref, dinv_ref, b_ref, wih_ref, bih_ref, bhh_ref,
               whh_ref, wf_ref, bf_ref, out_ref, xg_scr):
    h2 = jnp.maximum(
        dinv_ref[...] * (s_ref[0] + s_ref[1] + g_ref[...]) + b_ref[...],
        0.0)
    xg_scr[...] = (jnp.dot(h2, wih_ref[...], preferred_element_type=jnp.float32)
                   + bih_ref[...] + bhh_ref[...])
    whh = whh_ref[...]

    def cell(t, hp, cp):
        xt = xg_scr[pl.ds(t * B, B), :]
        gates = xt + jnp.dot(hp, whh, preferred_element_type=jnp.float32)
        i = jax.nn.sigmoid(gates[:, :H])
        f = jax.nn.sigmoid(gates[:, H:2 * H])
        g = jnp.tanh(gates[:, 2 * H:3 * H])
        o = jax.nn.sigmoid(gates[:, 3 * H:])
        c = f * cp + i * g
        hn = o * jnp.tanh(c)
        return hn, c

    def step(k, carry):
        hp, cp = carry
        hp, cp = cell(2 * k, hp, cp)
        hp, cp = cell(2 * k + 1, hp, cp)
        return (hp, cp)

    init = (jnp.zeros((B, H), jnp.float32), jnp.zeros((B, H), jnp.float32))
    hn, _ = lax.fori_loop(0, T // 2, step, init)
    out_ref[...] = jnp.dot(
        hn, wf_ref[...], preferred_element_type=jnp.float32) + bf_ref[...]


def _tc(body, out_shapes):
    return pl.pallas_call(body, out_shape=out_shapes)


@functools.cache
def _sc_kernels():
    mesh = plsc.VectorSubcoreMesh(
        core_axis_name="c", subcore_axis_name="s",
        num_cores=NC, num_subcores=NS)
    params = pltpu.CompilerParams(needs_layout_passes=False)
    deg = pl.kernel(
        _deg_body,
        out_type=(jax.ShapeDtypeStruct((NC, NS, H), jnp.float32),
                  jax.ShapeDtypeStruct((2, NW, NCH, CH), jnp.int32)),
        mesh=mesh,
        compiler_params=params,
        scratch_types=[
            pltpu.VMEM((NCH, CH), jnp.int32),
            pltpu.VMEM((NCH, CH), jnp.int32),
            pltpu.VMEM((NS, H), jnp.float32),
            pltpu.VMEM((1, 16), jnp.int32),
            pltpu.VMEM_SHARED((NS, H), jnp.float32),
        ],
    )
    scatter = pl.kernel(
        _scatter_body,
        out_type=jax.ShapeDtypeStruct((NC, N, H), jnp.float32),
        mesh=mesh,
        compiler_params=params,
        scratch_types=[
            pltpu.VMEM((NCH, CH), jnp.int32),
            pltpu.VMEM((NCH, CH), jnp.int32),
        ] + [pltpu.VMEM((CH, H), jnp.float32)] * NBUF + [
            pltpu.VMEM_SHARED((N, H), jnp.float32),
        ] + [pltpu.SemaphoreType.DMA] * (2 * NBUF),
    )
    return deg, scatter


def kernel(x, distance, edge_index, W0, b0, W1, b1, Wih, Whh, bih, bhh, Wf, bf):
    x2d = x.transpose(1, 0, 2).reshape(N, L)
    dflat = distance.reshape(1, L * L)
    edge_r = edge_index.reshape(2, NW, NCH, CH)
    zero_rows = jnp.zeros((RPT, H), jnp.float32)
    _deg_kernel, _scatter_kernel = _sc_kernels()

    dp, er = _deg_kernel(edge_r)
    dpcol = (dp[0] + dp[1]).reshape(N, 1)

    hw0 = _tc(_hw0_body, jax.ShapeDtypeStruct((N, H), jnp.float32))(
        x2d, dflat, W0)
    g0, dinv = _tc(_scale_body, [
        jax.ShapeDtypeStruct((N, H), jnp.float32),
        jax.ShapeDtypeStruct((N, 1), jnp.float32),
    ])(hw0, dpcol)

    s0 = _scatter_kernel(g0, er, zero_rows)
    g1 = _tc(_layer_body, jax.ShapeDtypeStruct((N, H), jnp.float32))(
        s0, g0, dinv, b0.reshape(1, H), W1)

    s1 = _scatter_kernel(g1, er, zero_rows)
    out = pl.pallas_call(
        _lstm_body,
        out_shape=jax.ShapeDtypeStruct((B, OUT), jnp.float32),
        scratch_shapes=[pltpu.VMEM((N, 4 * H), jnp.float32)],
    )(s1, g1, dinv, b1.reshape(1, H), Wih.T,
      bih.reshape(1, 4 * H), bhh.reshape(1, 4 * H),
      Whh.T, Wf, bf.reshape(1, OUT))
    return out.reshape(B, 12, L)
